# baseline (speedup 1.0000x reference)
import functools

import jax
import jax.numpy as jnp
from jax import lax
from jax.experimental import pallas as pl
from jax.experimental.pallas import tpu as pltpu
from jax.experimental.pallas import tpu_sc as plsc

D_EMB = 32
SCALE = float(D_EMB ** 0.5)
NC, NS = 2, 16
NW = NC * NS
D_VOC = 1000000

CW = 512
N_FULL = D_VOC // CW
TAIL = D_VOC - N_FULL * CW
N_PAIRS = 30

SB = 256
KG = SB // 128


def _mesh():
    return plsc.VectorSubcoreMesh(
        core_axis_name="c", subcore_axis_name="s",
        num_cores=NC, num_subcores=NS)


def _relayout(WT, tail16):
    @functools.partial(
        pl.kernel,
        out_type=jax.ShapeDtypeStruct((D_VOC // 4, 128), jnp.float32),
        mesh=_mesh(),
        scratch_types=(
            [pltpu.VMEM((D_EMB, CW), jnp.float32) for _ in range(2)]
            + [pltpu.VMEM((CW // 4, 129), jnp.float32) for _ in range(2)]
            + [pltpu.SemaphoreType.DMA for _ in range(4)]
        ),
        compiler_params=pltpu.CompilerParams(use_tc_tiling_on_sc=True, needs_layout_passes=False),
    )
    def body(wt_hbm, tail_hbm, out_hbm, ia, ib, oa, ob, gsa, gsb, osa, osb):
        wid = lax.axis_index("s") * NC + lax.axis_index("c")
        iota = lax.iota(jnp.int32, 16)

        def fire_in(c, ibuf, gsem):
            v0 = (c * NW + wid) * CW
            return pltpu.async_copy(wt_hbm.at[:, pl.ds(v0, CW)], ibuf, gsem)

        def wait_in(ibuf, gsem):
            pltpu.make_async_copy(
                wt_hbm.at[:, pl.ds(0, CW)], ibuf, gsem).wait()

        p_row = lax.shift_right_logical(iota, 2)
        p_col = (iota & 3) * D_EMB

        def transpose_chunk(ibuf, obuf):
            @plsc.parallel_loop(0, CW // 16, unroll=2)
            def blk(b):
                for e in range(D_EMB):
                    va = ibuf[e, pl.ds(16 * b, 16)]
                    plsc.store_scatter(
                        obuf, [4 * b + p_row, p_col + e], va * SCALE)

        def fire_out(c, obuf, osem):
            r0 = (c * NW + wid) * (CW // 4)
            return pltpu.async_copy(
                obuf.at[:, pl.ds(0, 128)], out_hbm.at[pl.ds(r0, CW // 4)],
                osem)

        def drain_out(osem):
            pltpu.make_async_copy(
                oa.at[:, pl.ds(0, 128)], out_hbm.at[pl.ds(0, CW // 4)],
                osem).wait()

        fire_in(0, ia, gsa)

        @pl.loop(0, N_PAIRS)
        def pair(p):
            fire_in(2 * p + 1, ib, gsb)
            wait_in(ia, gsa)

            @pl.when(p > 0)
            def _():
                drain_out(osa)
            transpose_chunk(ia, oa)
            fire_out(2 * p, oa, osa)
            fire_in(2 * p + 2, ia, gsa)
            wait_in(ib, gsb)

            @pl.when(p > 0)
            def _():
                drain_out(osb)
            transpose_chunk(ib, ob)
            fire_out(2 * p + 1, ob, osb)

        wait_in(ia, gsa)
        drain_out(osa)
        transpose_chunk(ia, oa)
        fire_out(2 * N_PAIRS, oa, osa)
        drain_out(osb)

        @pl.when(wid == 0)
        def _():
            cp = pltpu.async_copy(
                wt_hbm.at[:, pl.ds((N_FULL - 1) * CW, CW)], ib, gsb)
            cp.wait()
            transpose_chunk(ib, ob)
            cpo = pltpu.async_copy(
                ob.at[:, pl.ds(0, 128)],
                out_hbm.at[pl.ds((N_FULL - 1) * (CW // 4), CW // 4)],
                osb)
            cpo.wait()
            cpt = pltpu.async_copy(
                tail_hbm, ob.at[pl.ds(0, TAIL // 4), pl.ds(0, 128)], gsb)
            cpt.wait()
            cpt2 = pltpu.async_copy(
                ob.at[pl.ds(0, TAIL // 4), pl.ds(0, 128)],
                out_hbm.at[pl.ds(N_FULL * (CW // 4), TAIL // 4)], osa)
            cpt2.wait()

        drain_out(osa)

    return body(WT, tail16)


def _lookup(tok1d, T128, n_t, n_s):
    n_sblk = n_s // SB
    t_half = n_t // 2

    @functools.partial(
        pl.kernel,
        out_type=jax.ShapeDtypeStruct((n_t, D_EMB, n_s), jnp.float32),
        mesh=_mesh(),
        scratch_types=(
            [pltpu.VMEM((SB,), jnp.int32) for _ in range(4)]
            + [pltpu.VMEM((SB, 128), jnp.float32) for _ in range(2)]
            + [pltpu.VMEM((D_EMB, SB + 1), jnp.float32) for _ in range(2)]
            + [pltpu.SemaphoreType.DMA for _ in range(4)]
        ),
        compiler_params=pltpu.CompilerParams(use_tc_tiling_on_sc=True, needs_layout_passes=False),
    )
    def body(tok_hbm, w_hbm, out_hbm,
             iva, ivb, ova, ovb, ga, gb, oba, obb,
             gsa, gsb, osa, osb):
        wid = lax.axis_index("s") * NC + lax.axis_index("c")
        sblk = lax.rem(wid, n_sblk)
        t0 = lax.div(wid, n_sblk) * t_half
        s0 = sblk * SB
        iota = lax.iota(jnp.int32, 16)

        def stage_idx(t, iv, ov, gbuf, gsem):
            pltpu.sync_copy(tok_hbm.at[pl.ds(t * n_s + s0, SB)], iv)

            @pl.loop(0, SB // 16)
            def seg(i):
                v = iv[pl.ds(16 * i, 16)]
                ov[pl.ds(16 * i, 16)] = (v & 3) * D_EMB
                iv[pl.ds(16 * i, 16)] = lax.shift_right_logical(v, 2)
            for j in range(KG):
                pltpu.async_copy(
                    w_hbm.at[iv.at[pl.ds(128 * j, 128)]],
                    gbuf.at[pl.ds(128 * j, 128)], gsem)

        def drain_g(gbuf, gsem):
            for j in range(KG):
                pltpu.make_async_copy(
                    w_hbm.at[pl.ds(0, 128)],
                    gbuf.at[pl.ds(128 * j, 128)], gsem).wait()

        def extract(gbuf, ov, obuf):
            @plsc.parallel_loop(0, SB // 16, unroll=2)
            def grp(i):
                ov16 = ov[pl.ds(16 * i, 16)]
                for j in range(16):
                    s = 16 * i + j
                    o = pl.multiple_of(ov16[j], 16)
                    sv = jnp.full((16,), s, jnp.int32)
                    va = gbuf[s, pl.ds(o, 16)]
                    vb = gbuf[s, pl.ds(o + 16, 16)]
                    plsc.store_scatter(obuf, [iota, sv], va)
                    plsc.store_scatter(obuf, [16 + iota, sv], vb)

        def fire_out(t, obuf, osem):
            return pltpu.async_copy(
                obuf.at[:, pl.ds(0, SB)], out_hbm.at[t, :, pl.ds(s0, SB)],
                osem)

        def drain_out(obuf, osem):
            pltpu.make_async_copy(
                obuf.at[:, pl.ds(0, SB)], out_hbm.at[0, :, pl.ds(s0, SB)],
                osem).wait()

        stage_idx(t0, iva, ova, ga, gsa)

        @pl.loop(0, t_half // 2)
        def pair(p):
            ta = t0 + 2 * p
            stage_idx(ta + 1, ivb, ovb, gb, gsb)
            drain_g(ga, gsa)

            @pl.when(p > 0)
            def _():
                drain_out(oba, osa)
            extract(ga, ova, oba)
            fire_out(ta, oba, osa)

            @pl.when(p + 1 < t_half // 2)
            def _():
                stage_idx(ta + 2, iva, ova, ga, gsa)
            drain_g(gb, gsb)

            @pl.when(p > 0)
            def _():
                drain_out(obb, osb)
            extract(gb, ovb, obb)
            fire_out(ta + 1, obb, osb)

        drain_out(oba, osa)
        drain_out(obb, osb)

    return body(tok1d, T128)


def kernel(tokens, W):
    n_seq, n_tok = tokens.shape
    WT = jnp.transpose(W)
    tail16 = (W[N_FULL * CW:] * SCALE).reshape(TAIL // 4, 128)
    T128 = _relayout(WT, tail16)
    tok1d = jnp.transpose(tokens).reshape(-1).astype(jnp.int32)
    out_phys = _lookup(tok1d, T128, n_tok, n_seq)
    return jnp.transpose(out_phys, (2, 0, 1))


# speedup vs baseline: 1.2494x; 1.2494x over previous
import functools

import jax
import jax.numpy as jnp
from jax import lax
from jax.experimental import pallas as pl
from jax.experimental.pallas import tpu as pltpu
from jax.experimental.pallas import tpu_sc as plsc

D_EMB = 32
SCALE = float(D_EMB ** 0.5)
NC, NS = 2, 16
NW = NC * NS
D_VOC = 1000000

CW = 512
N_FULL = D_VOC // CW
TAIL = D_VOC - N_FULL * CW
N_PAIRS = 30

SB = 256
KG = SB // 128


def _mesh():
    return plsc.VectorSubcoreMesh(
        core_axis_name="c", subcore_axis_name="s",
        num_cores=NC, num_subcores=NS)


def _relayout(WT, tail16):
    @functools.partial(
        pl.kernel,
        out_type=jax.ShapeDtypeStruct((D_VOC // 4, 128), jnp.float32),
        mesh=_mesh(),
        scratch_types=(
            [pltpu.VMEM((D_EMB, CW + 1), jnp.float32) for _ in range(2)]
            + [pltpu.VMEM((CW // 4, 128), jnp.float32) for _ in range(2)]
            + [pltpu.SemaphoreType.DMA for _ in range(4)]
        ),
        compiler_params=pltpu.CompilerParams(use_tc_tiling_on_sc=True, needs_layout_passes=False),
    )
    def body(wt_hbm, tail_hbm, out_hbm, ia, ib, oa, ob, gsa, gsb, osa, osb):
        wid = lax.axis_index("s") * NC + lax.axis_index("c")
        iota = lax.iota(jnp.int32, 16)

        def fire_in(c, ibuf, gsem):
            v0 = (c * NW + wid) * CW
            return pltpu.async_copy(wt_hbm.at[:, pl.ds(v0, CW)],
                                    ibuf.at[:, pl.ds(0, CW)], gsem)

        def wait_in(ibuf, gsem):
            pltpu.make_async_copy(
                wt_hbm.at[:, pl.ds(0, CW)], ibuf.at[:, pl.ds(0, CW)],
                gsem).wait()

        def transpose_chunk(ibuf, obuf):
            @plsc.parallel_loop(0, CW // 4, unroll=4)
            def row(r):
                for g in range(8):
                    o, e0 = g // 2, (g % 2) * 16
                    vals = plsc.load_gather(
                        ibuf, [e0 + iota,
                               jnp.full((16,), 4 * r + o, jnp.int32)])
                    obuf[r, pl.ds(16 * g, 16)] = vals * SCALE

        def fire_out(c, obuf, osem):
            r0 = (c * NW + wid) * (CW // 4)
            return pltpu.async_copy(
                obuf, out_hbm.at[pl.ds(r0, CW // 4)], osem)

        def drain_out(osem):
            pltpu.make_async_copy(
                oa, out_hbm.at[pl.ds(0, CW // 4)], osem).wait()

        fire_in(0, ia, gsa)

        @pl.loop(0, N_PAIRS)
        def pair(p):
            fire_in(2 * p + 1, ib, gsb)
            wait_in(ia, gsa)

            @pl.when(p > 0)
            def _():
                drain_out(osa)
            transpose_chunk(ia, oa)
            fire_out(2 * p, oa, osa)
            fire_in(2 * p + 2, ia, gsa)
            wait_in(ib, gsb)

            @pl.when(p > 0)
            def _():
                drain_out(osb)
            transpose_chunk(ib, ob)
            fire_out(2 * p + 1, ob, osb)

        wait_in(ia, gsa)
        drain_out(osa)
        transpose_chunk(ia, oa)
        fire_out(2 * N_PAIRS, oa, osa)
        drain_out(osb)

        @pl.when(wid == 0)
        def _():
            cp = pltpu.async_copy(
                wt_hbm.at[:, pl.ds((N_FULL - 1) * CW, CW)],
                ib.at[:, pl.ds(0, CW)], gsb)
            cp.wait()
            transpose_chunk(ib, ob)
            cpo = pltpu.async_copy(
                ob, out_hbm.at[pl.ds((N_FULL - 1) * (CW // 4), CW // 4)],
                osb)
            cpo.wait()
            cpt = pltpu.async_copy(tail_hbm, ob.at[pl.ds(0, TAIL // 4)], gsb)
            cpt.wait()
            cpt2 = pltpu.async_copy(
                ob.at[pl.ds(0, TAIL // 4)],
                out_hbm.at[pl.ds(N_FULL * (CW // 4), TAIL // 4)], osa)
            cpt2.wait()

        drain_out(osa)

    return body(WT, tail16)


def _lookup(tok1d, T128, n_t, n_s):
    n_sblk = n_s // SB
    t_half = n_t // 2

    @functools.partial(
        pl.kernel,
        out_type=jax.ShapeDtypeStruct((n_t, D_EMB, n_s), jnp.float32),
        mesh=_mesh(),
        scratch_types=(
            [pltpu.VMEM((SB,), jnp.int32) for _ in range(4)]
            + [pltpu.VMEM((SB, 128), jnp.float32) for _ in range(2)]
            + [pltpu.VMEM((D_EMB, SB), jnp.float32) for _ in range(2)]
            + [pltpu.SemaphoreType.DMA for _ in range(4)]
        ),
        compiler_params=pltpu.CompilerParams(use_tc_tiling_on_sc=True, needs_layout_passes=False),
    )
    def body(tok_hbm, w_hbm, out_hbm,
             iva, ivb, ova, ovb, ga, gb, oba, obb,
             gsa, gsb, osa, osb):
        wid = lax.axis_index("s") * NC + lax.axis_index("c")
        sblk = lax.rem(wid, n_sblk)
        t0 = lax.div(wid, n_sblk) * t_half
        s0 = sblk * SB
        iota = lax.iota(jnp.int32, 16)

        def stage_idx(t, iv, ov, gbuf, gsem):
            pltpu.sync_copy(tok_hbm.at[pl.ds(t * n_s + s0, SB)], iv)

            @pl.loop(0, SB // 16)
            def seg(i):
                v = iv[pl.ds(16 * i, 16)]
                ov[pl.ds(16 * i, 16)] = (v & 3) * D_EMB
                iv[pl.ds(16 * i, 16)] = lax.shift_right_logical(v, 2)
            for j in range(KG):
                pltpu.async_copy(
                    w_hbm.at[iv.at[pl.ds(128 * j, 128)]],
                    gbuf.at[pl.ds(128 * j, 128)], gsem)

        def drain_g(gbuf, gsem):
            for j in range(KG):
                pltpu.make_async_copy(
                    w_hbm.at[pl.ds(0, 128)],
                    gbuf.at[pl.ds(128 * j, 128)], gsem).wait()

        def extract(gbuf, ov, obuf):
            @plsc.parallel_loop(0, SB // 16, unroll=2)
            def seg(i):
                srow = 16 * i + iota
                ocol = ov[pl.ds(16 * i, 16)]
                for e in range(D_EMB):
                    obuf[e, pl.ds(16 * i, 16)] = plsc.load_gather(
                        gbuf, [srow, ocol + e])

        def fire_out(t, obuf, osem):
            return pltpu.async_copy(
                obuf, out_hbm.at[t, :, pl.ds(s0, SB)], osem)

        def drain_out(obuf, osem):
            pltpu.make_async_copy(
                obuf, out_hbm.at[0, :, pl.ds(s0, SB)], osem).wait()

        stage_idx(t0, iva, ova, ga, gsa)

        @pl.loop(0, t_half // 2)
        def pair(p):
            ta = t0 + 2 * p
            stage_idx(ta + 1, ivb, ovb, gb, gsb)
            drain_g(ga, gsa)

            @pl.when(p > 0)
            def _():
                drain_out(oba, osa)
            extract(ga, ova, oba)
            fire_out(ta, oba, osa)

            @pl.when(p + 1 < t_half // 2)
            def _():
                stage_idx(ta + 2, iva, ova, ga, gsa)
            drain_g(gb, gsb)

            @pl.when(p > 0)
            def _():
                drain_out(obb, osb)
            extract(gb, ovb, obb)
            fire_out(ta + 1, obb, osb)

        drain_out(oba, osa)
        drain_out(obb, osb)

    return body(tok1d, T128)


def kernel(tokens, W):
    n_seq, n_tok = tokens.shape
    WT = jnp.transpose(W)
    tail16 = (W[N_FULL * CW:] * SCALE).reshape(TAIL // 4, 128)
    T128 = _relayout(WT, tail16)
    tok1d = jnp.transpose(tokens).reshape(-1).astype(jnp.int32)
    out_phys = _lookup(tok1d, T128, n_tok, n_seq)
    return jnp.transpose(out_phys, (2, 0, 1))


# speedup vs baseline: 1.8991x; 1.5201x over previous
import functools

import jax
import jax.numpy as jnp
from jax import lax
from jax.experimental import pallas as pl
from jax.experimental.pallas import tpu as pltpu
from jax.experimental.pallas import tpu_sc as plsc

D_EMB = 32
SCALE = float(D_EMB ** 0.5)
NC, NS = 2, 16
NW = NC * NS
D_VOC = 1000000

CW = 512
N_FULL = D_VOC // CW
TAIL = D_VOC - N_FULL * CW
N_PAIRS = 30

SB = 256
KG = SB // 128


def _mesh():
    return plsc.VectorSubcoreMesh(
        core_axis_name="c", subcore_axis_name="s",
        num_cores=NC, num_subcores=NS)


def _relayout(WT, tail16):
    @functools.partial(
        pl.kernel,
        out_type=jax.ShapeDtypeStruct((D_VOC // 4, 128), jnp.float32),
        mesh=_mesh(),
        scratch_types=(
            [pltpu.VMEM((D_EMB, CW + 1), jnp.float32) for _ in range(2)]
            + [pltpu.VMEM((CW // 4, 128), jnp.float32) for _ in range(2)]
            + [pltpu.SemaphoreType.DMA for _ in range(4)]
        ),
        compiler_params=pltpu.CompilerParams(use_tc_tiling_on_sc=True, needs_layout_passes=False),
    )
    def body(wt_hbm, tail_hbm, out_hbm, ia, ib, oa, ob, gsa, gsb, osa, osb):
        wid = lax.axis_index("s") * NC + lax.axis_index("c")
        iota = lax.iota(jnp.int32, 16)

        def fire_in(c, ibuf, gsem):
            v0 = (c * NW + wid) * CW
            return pltpu.async_copy(wt_hbm.at[:, pl.ds(v0, CW)],
                                    ibuf.at[:, pl.ds(0, CW)], gsem)

        def wait_in(ibuf, gsem):
            pltpu.make_async_copy(
                wt_hbm.at[:, pl.ds(0, CW)], ibuf.at[:, pl.ds(0, CW)],
                gsem).wait()

        rot_perm = {k: (iota + k) % 16 for d in (8, 4, 2, 1) for k in (d, -d)}
        sel_mask = {d: (iota & d) != 0 for d in (8, 4, 2, 1)}

        def transpose16(vs):
            vs = list(vs)
            for d in (8, 4, 2, 1):
                m = sel_mask[d]
                nvs = list(vs)
                for i in range(16):
                    if i & d:
                        continue
                    top, bot = vs[i], vs[i + d]
                    nvs[i] = jnp.where(
                        m, bot[rot_perm[-d]], top)
                    nvs[i + d] = jnp.where(
                        m, bot, top[rot_perm[d]])
                vs = nvs
            return vs

        def transpose_chunk(ibuf, obuf):
            @plsc.parallel_loop(0, CW // 16, unroll=1)
            def blk(b):
                for h in range(2):
                    vs = [ibuf[16 * h + k, pl.ds(16 * b, 16)] * SCALE
                          for k in range(16)]
                    ts = transpose16(vs)
                    for l in range(16):
                        obuf[4 * b + l // 4,
                             pl.ds((l % 4) * D_EMB + 16 * h, 16)] = ts[l]

        def fire_out(c, obuf, osem):
            r0 = (c * NW + wid) * (CW // 4)
            return pltpu.async_copy(
                obuf, out_hbm.at[pl.ds(r0, CW // 4)], osem)

        def drain_out(osem):
            pltpu.make_async_copy(
                oa, out_hbm.at[pl.ds(0, CW // 4)], osem).wait()

        fire_in(0, ia, gsa)

        @pl.loop(0, N_PAIRS)
        def pair(p):
            fire_in(2 * p + 1, ib, gsb)
            wait_in(ia, gsa)

            @pl.when(p > 0)
            def _():
                drain_out(osa)
            transpose_chunk(ia, oa)
            fire_out(2 * p, oa, osa)
            fire_in(2 * p + 2, ia, gsa)
            wait_in(ib, gsb)

            @pl.when(p > 0)
            def _():
                drain_out(osb)
            transpose_chunk(ib, ob)
            fire_out(2 * p + 1, ob, osb)

        wait_in(ia, gsa)
        drain_out(osa)
        transpose_chunk(ia, oa)
        fire_out(2 * N_PAIRS, oa, osa)
        drain_out(osb)

        @pl.when(wid == 0)
        def _():
            cp = pltpu.async_copy(
                wt_hbm.at[:, pl.ds((N_FULL - 1) * CW, CW)],
                ib.at[:, pl.ds(0, CW)], gsb)
            cp.wait()
            transpose_chunk(ib, ob)
            cpo = pltpu.async_copy(
                ob, out_hbm.at[pl.ds((N_FULL - 1) * (CW // 4), CW // 4)],
                osb)
            cpo.wait()
            cpt = pltpu.async_copy(tail_hbm, ob.at[pl.ds(0, TAIL // 4)], gsb)
            cpt.wait()
            cpt2 = pltpu.async_copy(
                ob.at[pl.ds(0, TAIL // 4)],
                out_hbm.at[pl.ds(N_FULL * (CW // 4), TAIL // 4)], osa)
            cpt2.wait()

        drain_out(osa)

    return body(WT, tail16)


def _lookup(tok1d, T128, n_t, n_s):
    n_sblk = n_s // SB
    t_half = n_t // 2

    @functools.partial(
        pl.kernel,
        out_type=jax.ShapeDtypeStruct((n_t, D_EMB, n_s), jnp.float32),
        mesh=_mesh(),
        scratch_types=(
            [pltpu.VMEM((SB,), jnp.int32) for _ in range(4)]
            + [pltpu.VMEM((SB, 128), jnp.float32) for _ in range(2)]
            + [pltpu.VMEM((D_EMB, SB), jnp.float32) for _ in range(2)]
            + [pltpu.SemaphoreType.DMA for _ in range(4)]
        ),
        compiler_params=pltpu.CompilerParams(use_tc_tiling_on_sc=True, needs_layout_passes=False),
    )
    def body(tok_hbm, w_hbm, out_hbm,
             iva, ivb, ova, ovb, ga, gb, oba, obb,
             gsa, gsb, osa, osb):
        wid = lax.axis_index("s") * NC + lax.axis_index("c")
        sblk = lax.rem(wid, n_sblk)
        t0 = lax.div(wid, n_sblk) * t_half
        s0 = sblk * SB
        iota = lax.iota(jnp.int32, 16)

        def stage_idx(t, iv, ov, gbuf, gsem):
            pltpu.sync_copy(tok_hbm.at[pl.ds(t * n_s + s0, SB)], iv)

            @pl.loop(0, SB // 16)
            def seg(i):
                v = iv[pl.ds(16 * i, 16)]
                ov[pl.ds(16 * i, 16)] = (v & 3) * D_EMB
                iv[pl.ds(16 * i, 16)] = lax.shift_right_logical(v, 2)
            for j in range(KG):
                pltpu.async_copy(
                    w_hbm.at[iv.at[pl.ds(128 * j, 128)]],
                    gbuf.at[pl.ds(128 * j, 128)], gsem)

        def drain_g(gbuf, gsem):
            for j in range(KG):
                pltpu.make_async_copy(
                    w_hbm.at[pl.ds(0, 128)],
                    gbuf.at[pl.ds(128 * j, 128)], gsem).wait()

        def extract(gbuf, ov, obuf):
            @plsc.parallel_loop(0, SB // 16, unroll=2)
            def seg(i):
                srow = 16 * i + iota
                ocol = ov[pl.ds(16 * i, 16)]
                for e in range(D_EMB):
                    obuf[e, pl.ds(16 * i, 16)] = plsc.load_gather(
                        gbuf, [srow, ocol + e])

        def fire_out(t, obuf, osem):
            return pltpu.async_copy(
                obuf, out_hbm.at[t, :, pl.ds(s0, SB)], osem)

        def drain_out(obuf, osem):
            pltpu.make_async_copy(
                obuf, out_hbm.at[0, :, pl.ds(s0, SB)], osem).wait()

        stage_idx(t0, iva, ova, ga, gsa)

        @pl.loop(0, t_half // 2)
        def pair(p):
            ta = t0 + 2 * p
            stage_idx(ta + 1, ivb, ovb, gb, gsb)
            drain_g(ga, gsa)

            @pl.when(p > 0)
            def _():
                drain_out(oba, osa)
            extract(ga, ova, oba)
            fire_out(ta, oba, osa)

            @pl.when(p + 1 < t_half // 2)
            def _():
                stage_idx(ta + 2, iva, ova, ga, gsa)
            drain_g(gb, gsb)

            @pl.when(p > 0)
            def _():
                drain_out(obb, osb)
            extract(gb, ovb, obb)
            fire_out(ta + 1, obb, osb)

        drain_out(oba, osa)
        drain_out(obb, osb)

    return body(tok1d, T128)


def kernel(tokens, W):
    n_seq, n_tok = tokens.shape
    WT = jnp.transpose(W)
    tail16 = (W[N_FULL * CW:] * SCALE).reshape(TAIL // 4, 128)
    T128 = _relayout(WT, tail16)
    tok1d = jnp.transpose(tokens).reshape(-1).astype(jnp.int32)
    out_phys = _lookup(tok1d, T128, n_tok, n_seq)
    return jnp.transpose(out_phys, (2, 0, 1))


# speedup vs baseline: 2.8672x; 1.5097x over previous
import functools

import jax
import jax.numpy as jnp
from jax import lax
from jax.experimental import pallas as pl
from jax.experimental.pallas import tpu as pltpu
from jax.experimental.pallas import tpu_sc as plsc

D_EMB = 32
SCALE = float(D_EMB ** 0.5)
NC, NS = 2, 16
NW = NC * NS
D_VOC = 1000000

CW = 512
N_FULL = D_VOC // CW
TAIL = D_VOC - N_FULL * CW
N_PAIRS = 30

SB = 256
KG = SB // 128


def _mesh():
    return plsc.VectorSubcoreMesh(
        core_axis_name="c", subcore_axis_name="s",
        num_cores=NC, num_subcores=NS)


def _perm_tables(iota):
    rot_perm = {k: (iota + k) % 16 for d in (8, 4, 2, 1) for k in (d, -d)}
    sel_mask = {d: (iota & d) != 0 for d in (8, 4, 2, 1)}
    return rot_perm, sel_mask


def _transpose16(vs, rot_perm, sel_mask):
    vs = list(vs)
    for d in (8, 4, 2, 1):
        m = sel_mask[d]
        nvs = list(vs)
        for i in range(16):
            if i & d:
                continue
            top, bot = vs[i], vs[i + d]
            nvs[i] = jnp.where(m, bot[rot_perm[-d]], top)
            nvs[i + d] = jnp.where(m, bot, top[rot_perm[d]])
        vs = nvs
    return vs


def _relayout(WT, tail16):
    @functools.partial(
        pl.kernel,
        out_type=jax.ShapeDtypeStruct((D_VOC // 4, 128), jnp.float32),
        mesh=_mesh(),
        scratch_types=(
            [pltpu.VMEM((D_EMB, CW + 1), jnp.float32) for _ in range(2)]
            + [pltpu.VMEM((CW // 4, 128), jnp.float32) for _ in range(2)]
            + [pltpu.SemaphoreType.DMA for _ in range(4)]
        ),
        compiler_params=pltpu.CompilerParams(use_tc_tiling_on_sc=True, needs_layout_passes=False),
    )
    def body(wt_hbm, tail_hbm, out_hbm, ia, ib, oa, ob, gsa, gsb, osa, osb):
        wid = lax.axis_index("s") * NC + lax.axis_index("c")
        iota = lax.iota(jnp.int32, 16)

        def fire_in(c, ibuf, gsem):
            v0 = (c * NW + wid) * CW
            return pltpu.async_copy(wt_hbm.at[:, pl.ds(v0, CW)],
                                    ibuf.at[:, pl.ds(0, CW)], gsem)

        def wait_in(ibuf, gsem):
            pltpu.make_async_copy(
                wt_hbm.at[:, pl.ds(0, CW)], ibuf.at[:, pl.ds(0, CW)],
                gsem).wait()

        rot_perm, sel_mask = _perm_tables(iota)

        def transpose_chunk(ibuf, obuf):
            @plsc.parallel_loop(0, CW // 16, unroll=1)
            def blk(b):
                for h in range(2):
                    vs = [ibuf[16 * h + k, pl.ds(16 * b, 16)] * SCALE
                          for k in range(16)]
                    ts = _transpose16(vs, rot_perm, sel_mask)
                    for l in range(16):
                        obuf[4 * b + l // 4,
                             pl.ds((l % 4) * D_EMB + 16 * h, 16)] = ts[l]

        def fire_out(c, obuf, osem):
            r0 = (c * NW + wid) * (CW // 4)
            return pltpu.async_copy(
                obuf, out_hbm.at[pl.ds(r0, CW // 4)], osem)

        def drain_out(osem):
            pltpu.make_async_copy(
                oa, out_hbm.at[pl.ds(0, CW // 4)], osem).wait()

        fire_in(0, ia, gsa)

        @pl.loop(0, N_PAIRS)
        def pair(p):
            fire_in(2 * p + 1, ib, gsb)
            wait_in(ia, gsa)

            @pl.when(p > 0)
            def _():
                drain_out(osa)
            transpose_chunk(ia, oa)
            fire_out(2 * p, oa, osa)
            fire_in(2 * p + 2, ia, gsa)
            wait_in(ib, gsb)

            @pl.when(p > 0)
            def _():
                drain_out(osb)
            transpose_chunk(ib, ob)
            fire_out(2 * p + 1, ob, osb)

        wait_in(ia, gsa)
        drain_out(osa)
        transpose_chunk(ia, oa)
        fire_out(2 * N_PAIRS, oa, osa)
        drain_out(osb)

        @pl.when(wid == 0)
        def _():
            cp = pltpu.async_copy(
                wt_hbm.at[:, pl.ds((N_FULL - 1) * CW, CW)],
                ib.at[:, pl.ds(0, CW)], gsb)
            cp.wait()
            transpose_chunk(ib, ob)
            cpo = pltpu.async_copy(
                ob, out_hbm.at[pl.ds((N_FULL - 1) * (CW // 4), CW // 4)],
                osb)
            cpo.wait()
            cpt = pltpu.async_copy(tail_hbm, ob.at[pl.ds(0, TAIL // 4)], gsb)
            cpt.wait()
            cpt2 = pltpu.async_copy(
                ob.at[pl.ds(0, TAIL // 4)],
                out_hbm.at[pl.ds(N_FULL * (CW // 4), TAIL // 4)], osa)
            cpt2.wait()

        drain_out(osa)

    return body(WT, tail16)


def _lookup(tok1d, T128, n_t, n_s):
    n_sblk = n_s // SB
    t_half = n_t // 2

    @functools.partial(
        pl.kernel,
        out_type=jax.ShapeDtypeStruct((n_t, D_EMB, n_s), jnp.float32),
        mesh=_mesh(),
        scratch_types=(
            [pltpu.VMEM((SB,), jnp.int32) for _ in range(2)]
            + [pltpu.VMEM((SB,), jnp.int32) for _ in range(2)]
            + [pltpu.VMEM((SB, 128), jnp.float32) for _ in range(2)]
            + [pltpu.VMEM((SB, D_EMB), jnp.float32)]
            + [pltpu.VMEM((D_EMB, SB), jnp.float32) for _ in range(2)]
            + [pltpu.SemaphoreType.DMA for _ in range(4)]
        ),
        compiler_params=pltpu.CompilerParams(use_tc_tiling_on_sc=True, needs_layout_passes=False),
    )
    def body(tok_hbm, w_hbm, out_hbm,
             iva, ivb, isa, isb, ga, gb, ca, oba, obb,
             gsa, gsb, osa, osb):
        wid = lax.axis_index("s") * NC + lax.axis_index("c")
        sblk = lax.rem(wid, n_sblk)
        t0 = lax.div(wid, n_sblk) * t_half
        s0 = sblk * SB
        iota = lax.iota(jnp.int32, 16)
        rot_perm, sel_mask = _perm_tables(iota)

        def stage_idx(t, iv, ivs, gbuf, gsem):
            pltpu.sync_copy(tok_hbm.at[pl.ds(t * n_s + s0, SB)], iv)

            @pl.loop(0, SB // 16)
            def seg(i):
                v = iv[pl.ds(16 * i, 16)]
                ivs[pl.ds(16 * i, 16)] = (v & 3) * D_EMB
                iv[pl.ds(16 * i, 16)] = lax.shift_right_logical(v, 2)
            for j in range(KG):
                pltpu.async_copy(
                    w_hbm.at[iv.at[pl.ds(128 * j, 128)]],
                    gbuf.at[pl.ds(128 * j, 128)], gsem)

        def drain_g(gbuf, gsem):
            for j in range(KG):
                pltpu.make_async_copy(
                    w_hbm.at[pl.ds(0, 128)],
                    gbuf.at[pl.ds(128 * j, 128)], gsem).wait()

        def extract(gbuf, ivs, cbuf, obuf):
            @plsc.parallel_loop(0, SB // 16, unroll=2)
            def grp0(i):
                o16 = ivs[pl.ds(16 * i, 16)]
                for j in range(16):
                    s = 16 * i + j
                    o = pl.multiple_of(o16[j], 16)
                    cbuf[s, pl.ds(0, 16)] = gbuf[s, pl.ds(o, 16)]
                    cbuf[s, pl.ds(16, 16)] = gbuf[s, pl.ds(o + 16, 16)]

            @plsc.parallel_loop(0, SB // 16, unroll=1)
            def grp(i):
                for h in range(2):
                    vs = [cbuf[16 * i + k, pl.ds(16 * h, 16)]
                          for k in range(16)]
                    ts = _transpose16(vs, rot_perm, sel_mask)
                    for l in range(16):
                        obuf[16 * h + l, pl.ds(16 * i, 16)] = ts[l]

        def fire_out(t, obuf, osem):
            return pltpu.async_copy(
                obuf, out_hbm.at[t, :, pl.ds(s0, SB)], osem)

        def drain_out(obuf, osem):
            pltpu.make_async_copy(
                obuf, out_hbm.at[0, :, pl.ds(s0, SB)], osem).wait()

        stage_idx(t0, iva, isa, ga, gsa)

        @pl.loop(0, t_half // 2)
        def pair(p):
            ta = t0 + 2 * p
            stage_idx(ta + 1, ivb, isb, gb, gsb)
            drain_g(ga, gsa)

            @pl.when(p > 0)
            def _():
                drain_out(oba, osa)
            extract(ga, isa, ca, oba)
            fire_out(ta, oba, osa)

            @pl.when(p + 1 < t_half // 2)
            def _():
                stage_idx(ta + 2, iva, isa, ga, gsa)
            drain_g(gb, gsb)

            @pl.when(p > 0)
            def _():
                drain_out(obb, osb)
            extract(gb, isb, ca, obb)
            fire_out(ta + 1, obb, osb)

        drain_out(oba, osa)
        drain_out(obb, osb)

    return body(tok1d, T128)


def kernel(tokens, W):
    n_seq, n_tok = tokens.shape
    WT = jnp.transpose(W)
    tail16 = (W[N_FULL * CW:] * SCALE).reshape(TAIL // 4, 128)
    T128 = _relayout(WT, tail16)
    tok1d = jnp.transpose(tokens).reshape(-1).astype(jnp.int32)
    out_phys = _lookup(tok1d, T128, n_tok, n_seq)
    return jnp.transpose(out_phys, (2, 0, 1))
